# baseline (speedup 1.0000x reference)
import functools

import jax
import jax.numpy as jnp
from jax.experimental import pallas as pl
from jax.experimental.pallas import tpu as pltpu

TB = 512
FT = 512


def _moe_dense_body(x_ref, wg_ref, w1_ref, w2_ref, out_ref, logits_ref, wall_ref):
    e = pl.program_id(1)
    f = pl.program_id(2)
    x = x_ref[...]

    @pl.when((e == 0) & (f == 0))
    def _init():
        logits = jnp.dot(x, wg_ref[...], preferred_element_type=jnp.float32)
        logits_ref[...] = logits
        p = jax.nn.softmax(logits, axis=-1)
        n_e = p.shape[1]
        iot = jax.lax.broadcasted_iota(jnp.int32, p.shape, 1)
        m1 = jnp.max(p, axis=-1, keepdims=True)
        i1 = jnp.min(jnp.where(p == m1, iot, n_e), axis=-1, keepdims=True)
        p2 = jnp.where(iot == i1, -1.0, p)
        m2 = jnp.max(p2, axis=-1, keepdims=True)
        i2 = jnp.min(jnp.where(p2 == m2, iot, n_e), axis=-1, keepdims=True)
        w = jnp.where(iot == i1, m1, jnp.where(iot == i2, m2, 0.0)) / (m1 + m2)
        wall_ref[...] = w
        out_ref[...] = jnp.zeros_like(out_ref)

    wall = wall_ref[...]
    lane = jax.lax.broadcasted_iota(jnp.int32, wall.shape, 1)
    w_e = jnp.sum(jnp.where(lane == e, wall, 0.0), axis=1, keepdims=True)
    h = jnp.maximum(
        jnp.dot(x, w1_ref[0], preferred_element_type=jnp.float32), 0.0)
    part = jnp.dot(h, w2_ref[0], preferred_element_type=jnp.float32)
    out_ref[...] += w_e * part


def _moe_dense(x, Wg, W1, W2):
    T, D = x.shape
    E = Wg.shape[1]
    F = W1.shape[2]
    grid = (T // TB, E, F // FT)
    out, logits = pl.pallas_call(
        _moe_dense_body,
        grid=grid,
        in_specs=[
            pl.BlockSpec((TB, D), lambda i, e, f: (i, 0)),
            pl.BlockSpec((D, E), lambda i, e, f: (0, 0)),
            pl.BlockSpec((1, D, FT), lambda i, e, f: (e, 0, f)),
            pl.BlockSpec((1, FT, D), lambda i, e, f: (e, f, 0)),
        ],
        out_specs=[
            pl.BlockSpec((TB, D), lambda i, e, f: (i, 0)),
            pl.BlockSpec((TB, E), lambda i, e, f: (i, 0)),
        ],
        out_shape=[
            jax.ShapeDtypeStruct((T, D), jnp.float32),
            jax.ShapeDtypeStruct((T, E), jnp.float32),
        ],
        scratch_shapes=[pltpu.VMEM((TB, 8), jnp.float32)],
    )(x, Wg, W1, W2)
    return out, logits


@jax.jit
def kernel(hidden_states, Wg, W1, W2):
    b, s, d = hidden_states.shape
    x = hidden_states.reshape(-1, d)
    out, logits = _moe_dense(x, Wg, W1, W2)
    return out.reshape(b, s, d), logits


# speedup vs baseline: 1.7085x; 1.7085x over previous
import functools

import jax
import jax.numpy as jnp
from jax import lax
from jax.experimental import pallas as pl
from jax.experimental.pallas import tpu as pltpu
from jax.experimental.pallas import tpu_sc as plsc

T, D, E, F = 4096, 1024, 8, 2048
BT = 256
NB = (2 * T) // BT + E
NS = NB * BT
TBR = 512
MB = 128
NW = 32
TPW = T // NW
CH = 32
CC = 16


def _router_body(x_ref, wg_ref, logits_ref, e0_ref, e1_ref, w0_ref, w1_ref,
                 cnt_ref):
    i = pl.program_id(0)
    x = x_ref[...]
    logits = jnp.dot(x, wg_ref[...], preferred_element_type=jnp.float32)
    logits_ref[...] = logits
    p = jax.nn.softmax(logits, axis=-1)
    iot = lax.broadcasted_iota(jnp.int32, p.shape, 1)
    m1 = jnp.max(p, axis=-1, keepdims=True)
    i1 = jnp.min(jnp.where(p == m1, iot, E), axis=-1, keepdims=True)
    p2 = jnp.where(iot == i1, -1.0, p)
    m2 = jnp.max(p2, axis=-1, keepdims=True)
    i2 = jnp.min(jnp.where(p2 == m2, iot, E), axis=-1, keepdims=True)
    e0_ref[...] = i1
    e1_ref[...] = i2
    denom = m1 + m2
    w0_ref[...] = jnp.broadcast_to(m1 / denom, (m1.shape[0], 16))
    w1_ref[...] = jnp.broadcast_to(m2 / denom, (m2.shape[0], 16))
    c0 = jnp.sum((iot == i1).astype(jnp.int32), axis=0, keepdims=True)
    c1 = jnp.sum((iot == i2).astype(jnp.int32), axis=0, keepdims=True)
    rows = lax.broadcasted_iota(jnp.int32, (8, E), 0)
    upd = jnp.where(rows == 0, c0, 0) + jnp.where(rows == 1, c1, 0)

    @pl.when(i == 0)
    def _():
        cnt_ref[...] = jnp.zeros_like(cnt_ref)

    cnt_ref[...] += upd


def _router(x, Wg):
    grid = (T // TBR,)
    return pl.pallas_call(
        _router_body,
        grid=grid,
        in_specs=[
            pl.BlockSpec((TBR, D), lambda i: (i, 0)),
            pl.BlockSpec((D, E), lambda i: (0, 0)),
        ],
        out_specs=[
            pl.BlockSpec((TBR, E), lambda i: (i, 0)),
            pl.BlockSpec((TBR, 1), lambda i: (i, 0)),
            pl.BlockSpec((TBR, 1), lambda i: (i, 0)),
            pl.BlockSpec((TBR, 16), lambda i: (i, 0)),
            pl.BlockSpec((TBR, 16), lambda i: (i, 0)),
            pl.BlockSpec((8, E), lambda i: (0, 0)),
        ],
        out_shape=[
            jax.ShapeDtypeStruct((T, E), jnp.float32),
            jax.ShapeDtypeStruct((T, 1), jnp.int32),
            jax.ShapeDtypeStruct((T, 1), jnp.int32),
            jax.ShapeDtypeStruct((T, 16), jnp.float32),
            jax.ShapeDtypeStruct((T, 16), jnp.float32),
            jax.ShapeDtypeStruct((8, E), jnp.int32),
        ],
    )(x, Wg)


def _meta_body(e0_ref, e1_ref, cnt_ref, pos0_ref, pos1_ref, be_ref,
               offp_ref, carry_ref):
    i = pl.program_id(0)

    @pl.when(i == 0)
    def _():
        c0 = cnt_ref[0:1, :].astype(jnp.float32)
        c1 = cnt_ref[1:2, :].astype(jnp.float32)
        total = c0 + c1
        padded = jnp.ceil(total / BT) * BT
        strict_lower = (lax.broadcasted_iota(jnp.int32, (E, E), 0) <
                        lax.broadcasted_iota(jnp.int32, (E, E), 1)
                        ).astype(jnp.float32)
        off = jnp.dot(padded, strict_lower,
                      preferred_element_type=jnp.float32)
        rows8 = lax.broadcasted_iota(jnp.int32, (8, E), 0)
        offp_ref[...] = jnp.where(rows8 == 0, off, 0.0)
        carry_ref[...] = jnp.where(rows8 == 1, c0, 0.0)
        bstart = lax.broadcasted_iota(jnp.int32, (8, NB), 1).astype(
            jnp.float32) * BT
        acc = jnp.zeros((8, NB), jnp.float32)
        lane8 = lax.broadcasted_iota(jnp.int32, (1, E), 1)
        for e in range(E):
            off_e = jnp.sum(jnp.where(lane8 == e, off, 0.0))
            acc += (off_e <= bstart).astype(jnp.float32)
        be_ref[...] = acc.astype(jnp.int32) - 1

    iot8 = lax.broadcasted_iota(jnp.int32, (MB, E), 1)
    strict_a = (lax.broadcasted_iota(jnp.int32, (MB, MB), 0) >
                lax.broadcasted_iota(jnp.int32, (MB, MB), 1)
                ).astype(jnp.float32)
    off_row = offp_ref[0:1, :]
    for crow, (eref, pref) in enumerate([(e0_ref, pos0_ref),
                                         (e1_ref, pos1_ref)]):
        m = (iot8 == eref[...]).astype(jnp.float32)
        exc = jnp.dot(strict_a, m, preferred_element_type=jnp.float32)
        carry = carry_ref[crow:crow + 1, :]
        slot = jnp.sum(m * (off_row + carry + exc), axis=1, keepdims=True)
        pref[...] = slot.astype(jnp.int32)
        carry_ref[crow:crow + 1, :] = carry + jnp.sum(m, axis=0, keepdims=True)


def _meta(e0, e1, cnt):
    grid = (T // MB,)
    return pl.pallas_call(
        _meta_body,
        grid=grid,
        in_specs=[
            pl.BlockSpec((MB, 1), lambda i: (i, 0)),
            pl.BlockSpec((MB, 1), lambda i: (i, 0)),
            pl.BlockSpec((8, E), lambda i: (0, 0)),
        ],
        out_specs=[
            pl.BlockSpec((MB, 1), lambda i: (i, 0)),
            pl.BlockSpec((MB, 1), lambda i: (i, 0)),
            pl.BlockSpec((8, NB), lambda i: (0, 0)),
        ],
        out_shape=[
            jax.ShapeDtypeStruct((T, 1), jnp.int32),
            jax.ShapeDtypeStruct((T, 1), jnp.int32),
            jax.ShapeDtypeStruct((8, NB), jnp.int32),
        ],
        scratch_shapes=[
            pltpu.VMEM((8, E), jnp.float32),
            pltpu.VMEM((8, E), jnp.float32),
        ],
    )(e0, e1, cnt)


def _make_dispatch():
    mesh = plsc.VectorSubcoreMesh(core_axis_name="c", subcore_axis_name="s")

    @functools.partial(
        pl.kernel,
        mesh=mesh,
        out_type=jax.ShapeDtypeStruct((NS, D), jnp.float32),
        scratch_types=[
            pltpu.VMEM((CH, D), jnp.float32),
            pltpu.VMEM((CH,), jnp.int32),
            pltpu.SemaphoreType.DMA,
        ],
    )
    def disp(x_hbm, pos0_hbm, pos1_hbm, xs_hbm, rows_v, idx_v, sem):
        wid = lax.axis_index("s") * 2 + lax.axis_index("c")
        base = wid * TPW
        for pos_hbm in (pos0_hbm, pos1_hbm):
            for c in range(TPW // CH):
                b = base + c * CH
                pltpu.sync_copy(pos_hbm.at[pl.ds(b, CH)], idx_v)
                pltpu.sync_copy(x_hbm.at[pl.ds(b, CH)], rows_v)
                pltpu.async_copy(rows_v, xs_hbm.at[idx_v], sem).wait()

    return disp


def _expert_body(be_ref, xs_ref, w1_ref, w2_ref, ys_ref):
    h = jnp.maximum(
        jnp.dot(xs_ref[...], w1_ref[0], preferred_element_type=jnp.float32),
        0.0)
    ys_ref[...] = jnp.dot(h, w2_ref[0], preferred_element_type=jnp.float32)


def _expert_mlp(be, xs, W1, W2):
    grid_spec = pltpu.PrefetchScalarGridSpec(
        num_scalar_prefetch=1,
        grid=(NB,),
        in_specs=[
            pl.BlockSpec((BT, D), lambda b, be_ref: (b, 0)),
            pl.BlockSpec((1, D, F), lambda b, be_ref: (be_ref[b], 0, 0)),
            pl.BlockSpec((1, F, D), lambda b, be_ref: (be_ref[b], 0, 0)),
        ],
        out_specs=pl.BlockSpec((BT, D), lambda b, be_ref: (b, 0)),
    )
    return pl.pallas_call(
        _expert_body,
        grid_spec=grid_spec,
        out_shape=jax.ShapeDtypeStruct((NS, D), jnp.float32),
    )(be, xs, W1, W2)


def _make_combine():
    mesh = plsc.VectorSubcoreMesh(core_axis_name="c", subcore_axis_name="s")

    @functools.partial(
        pl.kernel,
        mesh=mesh,
        out_type=jax.ShapeDtypeStruct((T, D), jnp.float32),
        scratch_types=[
            pltpu.VMEM((CC, D), jnp.float32),
            pltpu.VMEM((CC, D), jnp.float32),
            pltpu.VMEM((CC, D), jnp.float32),
            pltpu.VMEM((CC,), jnp.int32),
            pltpu.VMEM((CC,), jnp.int32),
            pltpu.VMEM((CC, 16), jnp.float32),
            pltpu.VMEM((CC, 16), jnp.float32),
            pltpu.SemaphoreType.DMA,
        ],
    )
    def comb(ys_hbm, pos0_hbm, pos1_hbm, w0_hbm, w1_hbm, out_hbm,
             b0, b1, ob, i0, i1, wv0, wv1, sem):
        wid = lax.axis_index("s") * 2 + lax.axis_index("c")
        base = wid * TPW
        for c in range(TPW // CC):
            b = base + c * CC
            pltpu.sync_copy(pos0_hbm.at[pl.ds(b, CC)], i0)
            pltpu.sync_copy(pos1_hbm.at[pl.ds(b, CC)], i1)
            pltpu.sync_copy(w0_hbm.at[pl.ds(b, CC)], wv0)
            pltpu.sync_copy(w1_hbm.at[pl.ds(b, CC)], wv1)
            pltpu.async_copy(ys_hbm.at[i0], b0, sem).wait()
            pltpu.async_copy(ys_hbm.at[i1], b1, sem).wait()

            def row(i, _):
                w0s = wv0[i, :]
                w1s = wv1[i, :]
                for j in range(D // 16):
                    a = b0[i, pl.ds(j * 16, 16)]
                    bb = b1[i, pl.ds(j * 16, 16)]
                    ob[i, pl.ds(j * 16, 16)] = w0s * a + w1s * bb
                return 0

            lax.fori_loop(0, CC, row, 0)
            pltpu.sync_copy(ob, out_hbm.at[pl.ds(b, CC)])

    return comb


@jax.jit
def kernel(hidden_states, Wg, W1, W2):
    b, s, d = hidden_states.shape
    x = hidden_states.reshape(-1, d)
    logits, e0, e1, w0, w1, cnt = _router(x, Wg)
    pos0, pos1, be2d = _meta(e0, e1, cnt)
    pos0f = pos0.reshape(T)
    pos1f = pos1.reshape(T)
    xs = _make_dispatch()(x, pos0f, pos1f)
    ys = _expert_mlp(be2d[0], xs, W1, W2)
    final = _make_combine()(ys, pos0f, pos1f, w0, w1)
    return final.reshape(b, s, d), logits


# speedup vs baseline: 1.9133x; 1.1198x over previous
import functools

import jax
import jax.numpy as jnp
from jax import lax
from jax.experimental import pallas as pl
from jax.experimental.pallas import tpu as pltpu
from jax.experimental.pallas import tpu_sc as plsc

T, D, E, F = 4096, 1024, 8, 2048
BT = 256
NB = (2 * T) // BT + E
NS = NB * BT
TBR = 512
MB = 128
NW = 32
TPW = T // NW
CH = 32
CC = 16


def _router_body(x_ref, wg_ref, logits_ref, e0_ref, e1_ref, w0_ref, w1_ref,
                 cnt_ref):
    i = pl.program_id(0)
    x = x_ref[...]
    logits = jnp.dot(x, wg_ref[...], preferred_element_type=jnp.float32)
    logits_ref[...] = logits
    p = jax.nn.softmax(logits, axis=-1)
    iot = lax.broadcasted_iota(jnp.int32, p.shape, 1)
    m1 = jnp.max(p, axis=-1, keepdims=True)
    i1 = jnp.min(jnp.where(p == m1, iot, E), axis=-1, keepdims=True)
    p2 = jnp.where(iot == i1, -1.0, p)
    m2 = jnp.max(p2, axis=-1, keepdims=True)
    i2 = jnp.min(jnp.where(p2 == m2, iot, E), axis=-1, keepdims=True)
    e0_ref[...] = i1
    e1_ref[...] = i2
    denom = m1 + m2
    w0_ref[...] = jnp.broadcast_to(m1 / denom, (m1.shape[0], 128))
    w1_ref[...] = jnp.broadcast_to(m2 / denom, (m2.shape[0], 128))
    c0 = jnp.sum((iot == i1).astype(jnp.int32), axis=0, keepdims=True)
    c1 = jnp.sum((iot == i2).astype(jnp.int32), axis=0, keepdims=True)
    rows = lax.broadcasted_iota(jnp.int32, (8, E), 0)
    upd = jnp.where(rows == 0, c0, 0) + jnp.where(rows == 1, c1, 0)

    @pl.when(i == 0)
    def _():
        cnt_ref[...] = jnp.zeros_like(cnt_ref)

    cnt_ref[...] += upd


def _router(x, Wg):
    grid = (T // TBR,)
    return pl.pallas_call(
        _router_body,
        grid=grid,
        in_specs=[
            pl.BlockSpec((TBR, D), lambda i: (i, 0)),
            pl.BlockSpec((D, E), lambda i: (0, 0)),
        ],
        out_specs=[
            pl.BlockSpec((TBR, E), lambda i: (i, 0)),
            pl.BlockSpec((TBR, 1), lambda i: (i, 0)),
            pl.BlockSpec((TBR, 1), lambda i: (i, 0)),
            pl.BlockSpec((TBR, 128), lambda i: (i, 0)),
            pl.BlockSpec((TBR, 128), lambda i: (i, 0)),
            pl.BlockSpec((8, E), lambda i: (0, 0)),
        ],
        out_shape=[
            jax.ShapeDtypeStruct((T, E), jnp.float32),
            jax.ShapeDtypeStruct((T, 1), jnp.int32),
            jax.ShapeDtypeStruct((T, 1), jnp.int32),
            jax.ShapeDtypeStruct((T, 128), jnp.float32),
            jax.ShapeDtypeStruct((T, 128), jnp.float32),
            jax.ShapeDtypeStruct((8, E), jnp.int32),
        ],
    )(x, Wg)


def _meta_body(e0_ref, e1_ref, cnt_ref, pos0_ref, pos1_ref, be_ref,
               offp_ref, carry_ref):
    i = pl.program_id(0)

    @pl.when(i == 0)
    def _():
        c0 = cnt_ref[0:1, :].astype(jnp.float32)
        c1 = cnt_ref[1:2, :].astype(jnp.float32)
        total = c0 + c1
        padded = jnp.ceil(total / BT) * BT
        strict_lower = (lax.broadcasted_iota(jnp.int32, (E, E), 0) <
                        lax.broadcasted_iota(jnp.int32, (E, E), 1)
                        ).astype(jnp.float32)
        off = jnp.dot(padded, strict_lower,
                      preferred_element_type=jnp.float32)
        rows8 = lax.broadcasted_iota(jnp.int32, (8, E), 0)
        offp_ref[...] = jnp.where(rows8 == 0, off, 0.0)
        carry_ref[...] = jnp.where(rows8 == 1, c0, 0.0)
        bstart = lax.broadcasted_iota(jnp.int32, (8, NB), 1).astype(
            jnp.float32) * BT
        acc = jnp.zeros((8, NB), jnp.float32)
        lane8 = lax.broadcasted_iota(jnp.int32, (1, E), 1)
        for e in range(E):
            off_e = jnp.sum(jnp.where(lane8 == e, off, 0.0))
            acc += (off_e <= bstart).astype(jnp.float32)
        be_ref[...] = acc.astype(jnp.int32) - 1

    iot8 = lax.broadcasted_iota(jnp.int32, (MB, E), 1)
    strict_a = (lax.broadcasted_iota(jnp.int32, (MB, MB), 0) >
                lax.broadcasted_iota(jnp.int32, (MB, MB), 1)
                ).astype(jnp.float32)
    off_row = offp_ref[0:1, :]
    for crow, (eref, pref) in enumerate([(e0_ref, pos0_ref),
                                         (e1_ref, pos1_ref)]):
        m = (iot8 == eref[...]).astype(jnp.float32)
        exc = jnp.dot(strict_a, m, preferred_element_type=jnp.float32)
        carry = carry_ref[crow:crow + 1, :]
        slot = jnp.sum(m * (off_row + carry + exc), axis=1, keepdims=True)
        pref[...] = slot.astype(jnp.int32)
        carry_ref[crow:crow + 1, :] = carry + jnp.sum(m, axis=0, keepdims=True)


def _meta(e0, e1, cnt):
    grid = (T // MB,)
    return pl.pallas_call(
        _meta_body,
        grid=grid,
        in_specs=[
            pl.BlockSpec((MB, 1), lambda i: (i, 0)),
            pl.BlockSpec((MB, 1), lambda i: (i, 0)),
            pl.BlockSpec((8, E), lambda i: (0, 0)),
        ],
        out_specs=[
            pl.BlockSpec((MB, 1), lambda i: (i, 0)),
            pl.BlockSpec((MB, 1), lambda i: (i, 0)),
            pl.BlockSpec((8, NB), lambda i: (0, 0)),
        ],
        out_shape=[
            jax.ShapeDtypeStruct((T, 1), jnp.int32),
            jax.ShapeDtypeStruct((T, 1), jnp.int32),
            jax.ShapeDtypeStruct((8, NB), jnp.int32),
        ],
        scratch_shapes=[
            pltpu.VMEM((8, E), jnp.float32),
            pltpu.VMEM((8, E), jnp.float32),
        ],
    )(e0, e1, cnt)


def _make_dispatch():
    mesh = plsc.VectorSubcoreMesh(core_axis_name="c", subcore_axis_name="s")
    nch = 2 * (TPW // CH)

    @functools.partial(
        pl.kernel,
        mesh=mesh,
        out_type=[
            jax.ShapeDtypeStruct((NS, D), jnp.float32),
            jax.ShapeDtypeStruct((NS, 128), jnp.float32),
        ],
        scratch_types=[
            pltpu.VMEM((CH, D), jnp.float32),
            pltpu.VMEM((CH, D), jnp.float32),
            pltpu.VMEM((CH, 128), jnp.float32),
            pltpu.VMEM((CH, 128), jnp.float32),
            pltpu.VMEM((CH,), jnp.int32),
            pltpu.VMEM((CH,), jnp.int32),
            pltpu.SemaphoreType.DMA,
            pltpu.SemaphoreType.DMA,
            pltpu.SemaphoreType.DMA,
            pltpu.SemaphoreType.DMA,
        ],
    )
    def disp(x_hbm, pos0_hbm, pos1_hbm, w0_hbm, w1_hbm, xs_hbm, ws_hbm,
             r0, r1, wc0, wc1, ix0, ix1, sl0, sl1, ss0, ss1):
        wid = lax.axis_index("s") * 2 + lax.axis_index("c")
        base = wid * TPW
        rows = (r0, r1)
        wcs = (wc0, wc1)
        idxs = (ix0, ix1)
        sls = (sl0, sl1)
        sss = (ss0, ss1)

        def src(i):
            k, c = divmod(i, TPW // CH)
            pos_hbm = pos0_hbm if k == 0 else pos1_hbm
            w_hbm = w0_hbm if k == 0 else w1_hbm
            return pos_hbm, w_hbm, base + c * CH

        def start_load(i, s):
            p, w, b = src(i)
            return (pltpu.async_copy(p.at[pl.ds(b, CH)], idxs[s], sls[s]),
                    pltpu.async_copy(x_hbm.at[pl.ds(b, CH)], rows[s], sls[s]),
                    pltpu.async_copy(w.at[pl.ds(b, CH)], wcs[s], sls[s]))

        loads = [None, None]
        scats = [None, None]
        loads[0] = start_load(0, 0)
        for i in range(nch):
            cur, nxt = i % 2, (i + 1) % 2
            for h in loads[cur]:
                h.wait()
            if scats[nxt] is not None:
                for h in scats[nxt]:
                    h.wait()
                scats[nxt] = None
            if i + 1 < nch:
                loads[nxt] = start_load(i + 1, nxt)
            scats[cur] = (
                pltpu.async_copy(rows[cur], xs_hbm.at[idxs[cur]], sss[cur]),
                pltpu.async_copy(wcs[cur], ws_hbm.at[idxs[cur]], sss[cur]))
        for s in scats:
            if s is not None:
                for h in s:
                    h.wait()

    return disp


def _expert_body(be_ref, xs_ref, w1_ref, w2_ref, ws_ref, ys_ref):
    xb = xs_ref[...].astype(jnp.bfloat16)
    w1b = w1_ref[0].astype(jnp.bfloat16)
    h = jnp.maximum(
        jnp.dot(xb, w1b, preferred_element_type=jnp.float32), 0.0)
    y = jnp.dot(h.astype(jnp.bfloat16), w2_ref[0].astype(jnp.bfloat16),
                preferred_element_type=jnp.float32)
    ys_ref[...] = y * ws_ref[:, 0:1]


def _expert_mlp(be, xs, W1, W2, ws):
    grid_spec = pltpu.PrefetchScalarGridSpec(
        num_scalar_prefetch=1,
        grid=(NB,),
        in_specs=[
            pl.BlockSpec((BT, D), lambda b, be_ref: (b, 0)),
            pl.BlockSpec((1, D, F), lambda b, be_ref: (be_ref[b], 0, 0)),
            pl.BlockSpec((1, F, D), lambda b, be_ref: (be_ref[b], 0, 0)),
            pl.BlockSpec((BT, 128), lambda b, be_ref: (b, 0)),
        ],
        out_specs=pl.BlockSpec((BT, D), lambda b, be_ref: (b, 0)),
    )
    return pl.pallas_call(
        _expert_body,
        grid_spec=grid_spec,
        out_shape=jax.ShapeDtypeStruct((NS, D), jnp.float32),
    )(be, xs, W1, W2, ws)


def _make_combine():
    mesh = plsc.VectorSubcoreMesh(core_axis_name="c", subcore_axis_name="s")
    nch = TPW // CC

    @functools.partial(
        pl.kernel,
        mesh=mesh,
        out_type=jax.ShapeDtypeStruct((T, D), jnp.float32),
        scratch_types=[
            pltpu.VMEM((CC, D), jnp.float32),
            pltpu.VMEM((CC, D), jnp.float32),
            pltpu.VMEM((CC, D), jnp.float32),
            pltpu.VMEM((CC, D), jnp.float32),
            pltpu.VMEM((CC,), jnp.int32),
            pltpu.VMEM((CC,), jnp.int32),
            pltpu.VMEM((CC,), jnp.int32),
            pltpu.VMEM((CC,), jnp.int32),
            pltpu.SemaphoreType.DMA,
            pltpu.SemaphoreType.DMA,
            pltpu.SemaphoreType.DMA,
            pltpu.SemaphoreType.DMA,
        ],
    )
    def comb(ys_hbm, pos0_hbm, pos1_hbm, out_hbm,
             b0a, b0b, b1a, b1b, i0a, i0b, i1a, i1b, sga, sgb, swa, swb):
        wid = lax.axis_index("s") * 2 + lax.axis_index("c")
        base = wid * TPW
        b0s, b1s = (b0a, b0b), (b1a, b1b)
        i0s, i1s = (i0a, i0b), (i1a, i1b)
        sgs, sws = (sga, sgb), (swa, swb)

        def start_chunk(i, s):
            b = base + i * CC
            pltpu.sync_copy(pos0_hbm.at[pl.ds(b, CC)], i0s[s])
            pltpu.sync_copy(pos1_hbm.at[pl.ds(b, CC)], i1s[s])
            return (pltpu.async_copy(ys_hbm.at[i0s[s]], b0s[s], sgs[s]),
                    pltpu.async_copy(ys_hbm.at[i1s[s]], b1s[s], sgs[s]))

        gath = [None, None]
        wrs = [None, None]
        gath[0] = start_chunk(0, 0)
        for i in range(nch):
            cur, nxt = i % 2, (i + 1) % 2
            for h in gath[cur]:
                h.wait()
            if wrs[nxt] is not None:
                wrs[nxt].wait()
                wrs[nxt] = None
            if i + 1 < nch:
                gath[nxt] = start_chunk(i + 1, nxt)
            b0, b1 = b0s[cur], b1s[cur]

            def row(r, _):
                for j in range(D // 16):
                    b0[r, pl.ds(j * 16, 16)] = (b0[r, pl.ds(j * 16, 16)] +
                                                b1[r, pl.ds(j * 16, 16)])
                return 0

            lax.fori_loop(0, CC, row, 0)
            wrs[cur] = pltpu.async_copy(
                b0, out_hbm.at[pl.ds(base + i * CC, CC)], sws[cur])
        for w_h in wrs:
            if w_h is not None:
                w_h.wait()

    return comb


@jax.jit
def kernel(hidden_states, Wg, W1, W2):
    b, s, d = hidden_states.shape
    x = hidden_states.reshape(-1, d)
    logits, e0, e1, w0, w1, cnt = _router(x, Wg)
    pos0, pos1, be2d = _meta(e0, e1, cnt)
    pos0f = pos0.reshape(T)
    pos1f = pos1.reshape(T)
    xs, ws = _make_dispatch()(x, pos0f, pos1f, w0, w1)
    ys = _expert_mlp(be2d[0], xs, W1, W2, ws)
    final = _make_combine()(ys, pos0f, pos1f)
    return final.reshape(b, s, d), logits


# speedup vs baseline: 2.0750x; 1.0845x over previous
import functools

import jax
import jax.numpy as jnp
from jax import lax
from jax.experimental import pallas as pl
from jax.experimental.pallas import tpu as pltpu
from jax.experimental.pallas import tpu_sc as plsc

T, D, E, F = 4096, 1024, 8, 2048
BT = 256
NB = (2 * T) // BT + E
NS = NB * BT
TBR = 512
MB = 512
NW = 32
TPW = T // NW
CH = 32
CC = 16


def _router_body(x_ref, wg_ref, logits_ref, e0_ref, e1_ref, w0_ref, w1_ref,
                 cnt_ref):
    i = pl.program_id(0)
    x = x_ref[...]
    logits = jnp.dot(x, wg_ref[...], preferred_element_type=jnp.float32)
    logits_ref[...] = logits
    p = jax.nn.softmax(logits, axis=-1)
    iot = lax.broadcasted_iota(jnp.int32, p.shape, 1)
    m1 = jnp.max(p, axis=-1, keepdims=True)
    i1 = jnp.min(jnp.where(p == m1, iot, E), axis=-1, keepdims=True)
    p2 = jnp.where(iot == i1, -1.0, p)
    m2 = jnp.max(p2, axis=-1, keepdims=True)
    i2 = jnp.min(jnp.where(p2 == m2, iot, E), axis=-1, keepdims=True)
    e0_ref[...] = i1
    e1_ref[...] = i2
    denom = m1 + m2
    w0_ref[...] = jnp.broadcast_to(m1 / denom, (m1.shape[0], 128))
    w1_ref[...] = jnp.broadcast_to(m2 / denom, (m2.shape[0], 128))
    c0 = jnp.sum((iot == i1).astype(jnp.int32), axis=0, keepdims=True)
    c1 = jnp.sum((iot == i2).astype(jnp.int32), axis=0, keepdims=True)
    rows = lax.broadcasted_iota(jnp.int32, (8, E), 0)
    upd = jnp.where(rows == 0, c0, 0) + jnp.where(rows == 1, c1, 0)

    @pl.when(i == 0)
    def _():
        cnt_ref[...] = jnp.zeros_like(cnt_ref)

    cnt_ref[...] += upd


def _router(x, Wg):
    grid = (T // TBR,)
    return pl.pallas_call(
        _router_body,
        grid=grid,
        in_specs=[
            pl.BlockSpec((TBR, D), lambda i: (i, 0)),
            pl.BlockSpec((D, E), lambda i: (0, 0)),
        ],
        out_specs=[
            pl.BlockSpec((TBR, E), lambda i: (i, 0)),
            pl.BlockSpec((TBR, 1), lambda i: (i, 0)),
            pl.BlockSpec((TBR, 1), lambda i: (i, 0)),
            pl.BlockSpec((TBR, 128), lambda i: (i, 0)),
            pl.BlockSpec((TBR, 128), lambda i: (i, 0)),
            pl.BlockSpec((8, E), lambda i: (0, 0)),
        ],
        out_shape=[
            jax.ShapeDtypeStruct((T, E), jnp.float32),
            jax.ShapeDtypeStruct((T, 1), jnp.int32),
            jax.ShapeDtypeStruct((T, 1), jnp.int32),
            jax.ShapeDtypeStruct((T, 128), jnp.float32),
            jax.ShapeDtypeStruct((T, 128), jnp.float32),
            jax.ShapeDtypeStruct((8, E), jnp.int32),
        ],
    )(x, Wg)


def _meta_body(e0_ref, e1_ref, cnt_ref, tri_ref, pos0_ref, pos1_ref, be_ref,
               offp_ref, carry_ref):
    i = pl.program_id(0)

    @pl.when(i == 0)
    def _():
        c0 = cnt_ref[0:1, :].astype(jnp.float32)
        c1 = cnt_ref[1:2, :].astype(jnp.float32)
        total = c0 + c1
        padded = jnp.ceil(total / BT) * BT
        strict_lower = (lax.broadcasted_iota(jnp.int32, (E, E), 0) <
                        lax.broadcasted_iota(jnp.int32, (E, E), 1)
                        ).astype(jnp.float32)
        off = jnp.dot(padded, strict_lower,
                      preferred_element_type=jnp.float32)
        rows8 = lax.broadcasted_iota(jnp.int32, (8, E), 0)
        offp_ref[...] = jnp.where(rows8 == 0, off, 0.0)
        carry_ref[...] = jnp.where(rows8 == 1, c0, 0.0)
        bstart = lax.broadcasted_iota(jnp.int32, (8, NB), 1).astype(
            jnp.float32) * BT
        acc = jnp.zeros((8, NB), jnp.float32)
        lane8 = lax.broadcasted_iota(jnp.int32, (1, E), 1)
        for e in range(E):
            off_e = jnp.sum(jnp.where(lane8 == e, off, 0.0))
            acc += (off_e <= bstart).astype(jnp.float32)
        be_ref[...] = acc.astype(jnp.int32) - 1

    iot8 = lax.broadcasted_iota(jnp.int32, (MB, E), 1)
    strict_a = tri_ref[...]
    off_row = offp_ref[0:1, :]
    for crow, (eref, pref) in enumerate([(e0_ref, pos0_ref),
                                         (e1_ref, pos1_ref)]):
        m = (iot8 == eref[...]).astype(jnp.float32)
        exc = jnp.dot(strict_a, m, preferred_element_type=jnp.float32)
        carry = carry_ref[crow:crow + 1, :]
        slot = jnp.sum(m * (off_row + carry + exc), axis=1, keepdims=True)
        pref[...] = slot.astype(jnp.int32)
        carry_ref[crow:crow + 1, :] = carry + jnp.sum(m, axis=0, keepdims=True)


def _meta(e0, e1, cnt):
    tri = (lax.broadcasted_iota(jnp.int32, (MB, MB), 0) >
           lax.broadcasted_iota(jnp.int32, (MB, MB), 1)).astype(jnp.float32)
    grid = (T // MB,)
    return pl.pallas_call(
        _meta_body,
        grid=grid,
        in_specs=[
            pl.BlockSpec((MB, 1), lambda i: (i, 0)),
            pl.BlockSpec((MB, 1), lambda i: (i, 0)),
            pl.BlockSpec((8, E), lambda i: (0, 0)),
            pl.BlockSpec((MB, MB), lambda i: (0, 0)),
        ],
        out_specs=[
            pl.BlockSpec((MB, 1), lambda i: (i, 0)),
            pl.BlockSpec((MB, 1), lambda i: (i, 0)),
            pl.BlockSpec((8, NB), lambda i: (0, 0)),
        ],
        out_shape=[
            jax.ShapeDtypeStruct((T, 1), jnp.int32),
            jax.ShapeDtypeStruct((T, 1), jnp.int32),
            jax.ShapeDtypeStruct((8, NB), jnp.int32),
        ],
        scratch_shapes=[
            pltpu.VMEM((8, E), jnp.float32),
            pltpu.VMEM((8, E), jnp.float32),
        ],
    )(e0, e1, cnt, tri)


def _make_dispatch():
    mesh = plsc.VectorSubcoreMesh(core_axis_name="c", subcore_axis_name="s")
    nch = TPW // CH

    @functools.partial(
        pl.kernel,
        mesh=mesh,
        out_type=[
            jax.ShapeDtypeStruct((NS, D), jnp.float32),
            jax.ShapeDtypeStruct((NS, 128), jnp.float32),
        ],
        scratch_types=[
            pltpu.VMEM((CH, D), jnp.float32),
            pltpu.VMEM((CH, D), jnp.float32),
            pltpu.VMEM((CH, 128), jnp.float32),
            pltpu.VMEM((CH, 128), jnp.float32),
            pltpu.VMEM((CH, 128), jnp.float32),
            pltpu.VMEM((CH, 128), jnp.float32),
            pltpu.VMEM((CH,), jnp.int32),
            pltpu.VMEM((CH,), jnp.int32),
            pltpu.VMEM((CH,), jnp.int32),
            pltpu.VMEM((CH,), jnp.int32),
            pltpu.SemaphoreType.DMA,
            pltpu.SemaphoreType.DMA,
            pltpu.SemaphoreType.DMA,
            pltpu.SemaphoreType.DMA,
        ],
    )
    def disp(x_hbm, pos0_hbm, pos1_hbm, w0_hbm, w1_hbm, xs_hbm, ws_hbm,
             r0, r1, wc0a, wc0b, wc1a, wc1b, ix0a, ix0b, ix1a, ix1b,
             sl0, sl1, ss0, ss1):
        wid = lax.axis_index("s") * 2 + lax.axis_index("c")
        base = wid * TPW
        rows = (r0, r1)
        wc0s, wc1s = (wc0a, wc0b), (wc1a, wc1b)
        ix0s, ix1s = (ix0a, ix0b), (ix1a, ix1b)
        sls = (sl0, sl1)
        sss = (ss0, ss1)

        def start_load(i, s):
            b = base + i * CH
            return (
                pltpu.async_copy(pos0_hbm.at[pl.ds(b, CH)], ix0s[s], sls[s]),
                pltpu.async_copy(pos1_hbm.at[pl.ds(b, CH)], ix1s[s], sls[s]),
                pltpu.async_copy(x_hbm.at[pl.ds(b, CH)], rows[s], sls[s]),
                pltpu.async_copy(w0_hbm.at[pl.ds(b, CH)], wc0s[s], sls[s]),
                pltpu.async_copy(w1_hbm.at[pl.ds(b, CH)], wc1s[s], sls[s]))

        loads = [None, None]
        scats = [None, None]
        loads[0] = start_load(0, 0)
        for i in range(nch):
            cur, nxt = i % 2, (i + 1) % 2
            for h in loads[cur]:
                h.wait()
            if scats[nxt] is not None:
                for h in scats[nxt]:
                    h.wait()
                scats[nxt] = None
            if i + 1 < nch:
                loads[nxt] = start_load(i + 1, nxt)
            scats[cur] = (
                pltpu.async_copy(rows[cur], xs_hbm.at[ix0s[cur]], sss[cur]),
                pltpu.async_copy(rows[cur], xs_hbm.at[ix1s[cur]], sss[cur]),
                pltpu.async_copy(wc0s[cur], ws_hbm.at[ix0s[cur]], sss[cur]),
                pltpu.async_copy(wc1s[cur], ws_hbm.at[ix1s[cur]], sss[cur]))
        for s in scats:
            if s is not None:
                for h in s:
                    h.wait()

    return disp


def _expert_body(be_ref, xs_ref, w1_ref, w2_ref, ws_ref, ys_ref):
    xb = xs_ref[...].astype(jnp.bfloat16)
    w1b = w1_ref[0].astype(jnp.bfloat16)
    h = jnp.maximum(
        jnp.dot(xb, w1b, preferred_element_type=jnp.float32), 0.0)
    y = jnp.dot(h.astype(jnp.bfloat16), w2_ref[0].astype(jnp.bfloat16),
                preferred_element_type=jnp.float32)
    ys_ref[...] = y * ws_ref[:, 0:1]


def _expert_mlp(be, xs, W1, W2, ws):
    grid_spec = pltpu.PrefetchScalarGridSpec(
        num_scalar_prefetch=1,
        grid=(NB,),
        in_specs=[
            pl.BlockSpec((BT, D), lambda b, be_ref: (b, 0)),
            pl.BlockSpec((1, D, F), lambda b, be_ref: (be_ref[b], 0, 0)),
            pl.BlockSpec((1, F, D), lambda b, be_ref: (be_ref[b], 0, 0)),
            pl.BlockSpec((BT, 128), lambda b, be_ref: (b, 0)),
        ],
        out_specs=pl.BlockSpec((BT, D), lambda b, be_ref: (b, 0)),
    )
    return pl.pallas_call(
        _expert_body,
        grid_spec=grid_spec,
        out_shape=jax.ShapeDtypeStruct((NS, D), jnp.float32),
    )(be, xs, W1, W2, ws)


def _make_combine():
    mesh = plsc.VectorSubcoreMesh(core_axis_name="c", subcore_axis_name="s")
    nch = TPW // CC

    @functools.partial(
        pl.kernel,
        mesh=mesh,
        out_type=jax.ShapeDtypeStruct((T, D), jnp.float32),
        scratch_types=[
            pltpu.VMEM((CC, D), jnp.float32),
            pltpu.VMEM((CC, D), jnp.float32),
            pltpu.VMEM((CC, D), jnp.float32),
            pltpu.VMEM((CC, D), jnp.float32),
            pltpu.VMEM((CC,), jnp.int32),
            pltpu.VMEM((CC,), jnp.int32),
            pltpu.VMEM((CC,), jnp.int32),
            pltpu.VMEM((CC,), jnp.int32),
            pltpu.SemaphoreType.DMA,
            pltpu.SemaphoreType.DMA,
            pltpu.SemaphoreType.DMA,
            pltpu.SemaphoreType.DMA,
        ],
    )
    def comb(ys_hbm, pos0_hbm, pos1_hbm, out_hbm,
             b0a, b0b, b1a, b1b, i0a, i0b, i1a, i1b, sga, sgb, swa, swb):
        wid = lax.axis_index("s") * 2 + lax.axis_index("c")
        base = wid * TPW
        b0s, b1s = (b0a, b0b), (b1a, b1b)
        i0s, i1s = (i0a, i0b), (i1a, i1b)
        sgs, sws = (sga, sgb), (swa, swb)

        def start_chunk(i, s):
            b = base + i * CC
            pltpu.sync_copy(pos0_hbm.at[pl.ds(b, CC)], i0s[s])
            pltpu.sync_copy(pos1_hbm.at[pl.ds(b, CC)], i1s[s])
            return (pltpu.async_copy(ys_hbm.at[i0s[s]], b0s[s], sgs[s]),
                    pltpu.async_copy(ys_hbm.at[i1s[s]], b1s[s], sgs[s]))

        gath = [None, None]
        wrs = [None, None]
        gath[0] = start_chunk(0, 0)
        for i in range(nch):
            cur, nxt = i % 2, (i + 1) % 2
            for h in gath[cur]:
                h.wait()
            if wrs[nxt] is not None:
                wrs[nxt].wait()
                wrs[nxt] = None
            if i + 1 < nch:
                gath[nxt] = start_chunk(i + 1, nxt)
            b0, b1 = b0s[cur], b1s[cur]

            def row(r, _):
                for j in range(D // 16):
                    b0[r, pl.ds(j * 16, 16)] = (b0[r, pl.ds(j * 16, 16)] +
                                                b1[r, pl.ds(j * 16, 16)])
                return 0

            lax.fori_loop(0, CC, row, 0)
            wrs[cur] = pltpu.async_copy(
                b0, out_hbm.at[pl.ds(base + i * CC, CC)], sws[cur])
        for w_h in wrs:
            if w_h is not None:
                w_h.wait()

    return comb


@jax.jit
def kernel(hidden_states, Wg, W1, W2):
    b, s, d = hidden_states.shape
    x = hidden_states.reshape(-1, d)
    logits, e0, e1, w0, w1, cnt = _router(x, Wg)
    pos0, pos1, be2d = _meta(e0, e1, cnt)
    pos0f = pos0.reshape(T)
    pos1f = pos1.reshape(T)
    xs, ws = _make_dispatch()(x, pos0f, pos1f, w0, w1)
    ys = _expert_mlp(be2d[0], xs, W1, W2, ws)
    final = _make_combine()(ys, pos0f, pos1f)
    return final.reshape(b, s, d), logits
